# Initial kernel scaffold; baseline (speedup 1.0000x reference)
import jax
import jax.numpy as jnp
from jax.experimental import pallas as pl


def kernel(input_ids, rel_ids, seq_table, chain_table, depth_table, role_table):
    raise NotImplementedError("write your pallas kernel here")


# speedup vs baseline: 15.4111x; 15.4111x over previous
import functools

import jax
import jax.numpy as jnp
from jax import lax
from jax.experimental import pallas as pl
from jax.experimental.pallas import tpu as pltpu
from jax.experimental.pallas import tpu_sc as plsc


def _fused_index_body(chain_rows, input_ref, rel_ref, out_ref):
    inp = input_ref[...]
    rel = rel_ref[...]
    Bn, Tn = inp.shape
    t = lax.broadcasted_iota(jnp.int32, (Bn, Tn), 1)
    m = rel > 0
    z = jnp.where(m, -1, t)
    k = 1
    while k < Tn:
        shifted = jnp.concatenate(
            [jnp.full((Bn, k), -1, jnp.int32), z[:, :-k]], axis=1)
        z = jnp.maximum(z, shifted)
        k *= 2
    cp = jnp.where(m, t - z - 1, 0)
    cp = jnp.minimum(cp, chain_rows - 1)
    special = (inp >= 0) & (inp <= 4)
    rcode = jnp.where(special, 2, jnp.where(rel == 0, 1, 0))
    out_ref[...] = cp * 3 + rcode


def _fused_indices(input_ids, rel_ids, chain_rows):
    Bn, Tn = input_ids.shape
    return pl.pallas_call(
        functools.partial(_fused_index_body, chain_rows),
        out_shape=jax.ShapeDtypeStruct((Bn, Tn), jnp.int32),
    )(input_ids, rel_ids)


def _m_table_body(chain_ref, combo_ref, m_ref):
    c = 0.5 * chain_ref[...]
    for r in range(3):
        m_ref[:, r, :] = c + combo_ref[r : r + 1, :]


def _m_table(chain_table, combo):
    V, D = chain_table.shape
    CB = 200 if V % 200 == 0 else V
    m3 = pl.pallas_call(
        _m_table_body,
        grid=(V // CB,),
        in_specs=[
            pl.BlockSpec((CB, D), lambda i: (i, 0)),
            pl.BlockSpec((3, D), lambda i: (0, 0)),
        ],
        out_specs=pl.BlockSpec((CB, 3, D), lambda i: (i, 0, 0)),
        out_shape=jax.ShapeDtypeStruct((V, 3, D), jnp.float32),
    )(chain_table, combo)
    return m3.reshape(V * 3, D)


def _sc_combine(seq_table, m_tab, fidx_flat, N, T, D, tok_w, K):
    L = 16
    nvec = D // L
    nchunk = tok_w // K
    mesh = plsc.VectorSubcoreMesh(core_axis_name="c", subcore_axis_name="s")
    num_cores = 2

    @functools.partial(
        pl.kernel,
        mesh=mesh,
        out_type=jax.ShapeDtypeStruct((N, D), jnp.float32),
        scratch_types=[
            pltpu.VMEM((tok_w,), jnp.int32),
            pltpu.VMEM((K, D), jnp.float32),
            pltpu.VMEM((K, D), jnp.float32),
            pltpu.SemaphoreType.DMA,
        ],
    )
    def sc_kernel(seq_hbm, m_hbm, fidx_hbm, out_hbm, idx_v, seq_v, acc_v, sem):
        wid = lax.axis_index("s") * num_cores + lax.axis_index("c")
        base = wid * tok_w
        t0 = lax.rem(base, T)
        pltpu.sync_copy(fidx_hbm.at[pl.ds(base, tok_w)], idx_v)

        def chunk_body(c, _):
            off = c * K
            cp = pltpu.async_copy(
                m_hbm.at[idx_v.at[pl.ds(off, K)]], acc_v, sem)
            pltpu.sync_copy(seq_hbm.at[pl.ds(t0 + off, K)], seq_v)
            cp.wait()

            def row_body(i, _):
                def vec_body(j, _):
                    sl = pl.ds(j * L, L)
                    acc_v[i, sl] = acc_v[i, sl] + seq_v[i, sl]
                    return 0

                lax.fori_loop(0, nvec, vec_body, 0, unroll=8)
                return 0

            lax.fori_loop(0, K, row_body, 0)
            pltpu.sync_copy(acc_v, out_hbm.at[pl.ds(base + off, K)])
            return 0

        lax.fori_loop(0, nchunk, chunk_body, 0)

    return sc_kernel(seq_table, m_tab, fidx_flat)


def kernel(input_ids, rel_ids, seq_table, chain_table, depth_table, role_table):
    Bn, Tn = input_ids.shape
    Tseq, D = seq_table.shape
    V = chain_table.shape[0]
    N = Bn * Tn

    role3 = jnp.concatenate(
        [role_table[0:1], role_table[2:3], role_table[3:4]], axis=0)
    combo = 0.3 * depth_table[0:1] + 0.2 * role3

    fidx = _fused_indices(input_ids, rel_ids, V)
    m_tab = _m_table(chain_table, combo)

    num_workers = 32
    tok_w = N // num_workers
    K = 32
    out = _sc_combine(
        seq_table, m_tab, fidx.reshape(N), N, Tn, D, tok_w, K)
    return out.reshape(Bn, Tn, D)


# speedup vs baseline: 18.8206x; 1.2212x over previous
import functools

import jax
import jax.numpy as jnp
from jax import lax
from jax.experimental import pallas as pl
from jax.experimental.pallas import tpu as pltpu
from jax.experimental.pallas import tpu_sc as plsc


def _fused_index_body(chain_rows, input_ref, rel_ref, out_ref):
    inp = input_ref[...]
    rel = rel_ref[...]
    Bn, Tn = inp.shape
    t = lax.broadcasted_iota(jnp.int32, (Bn, Tn), 1)
    m = rel > 0
    z = jnp.where(m, -1, t)
    k = 1
    while k < Tn:
        shifted = jnp.concatenate(
            [jnp.full((Bn, k), -1, jnp.int32), z[:, :-k]], axis=1)
        z = jnp.maximum(z, shifted)
        k *= 2
    cp = jnp.where(m, t - z - 1, 0)
    cp = jnp.minimum(cp, chain_rows - 1)
    special = (inp >= 0) & (inp <= 4)
    rcode = jnp.where(special, 2, jnp.where(rel == 0, 1, 0))
    out_ref[...] = cp * 3 + rcode


def _fused_indices(input_ids, rel_ids, chain_rows):
    Bn, Tn = input_ids.shape
    return pl.pallas_call(
        functools.partial(_fused_index_body, chain_rows),
        out_shape=jax.ShapeDtypeStruct((Bn, Tn), jnp.int32),
    )(input_ids, rel_ids)


def _m_table_body(chain_ref, combo_ref, m_ref):
    c = 0.5 * chain_ref[...]
    for r in range(3):
        m_ref[:, r, :] = c + combo_ref[r : r + 1, :]


def _m_table(chain_table, combo):
    V, D = chain_table.shape
    CB = 200 if V % 200 == 0 else V
    m3 = pl.pallas_call(
        _m_table_body,
        grid=(V // CB,),
        in_specs=[
            pl.BlockSpec((CB, D), lambda i: (i, 0)),
            pl.BlockSpec((3, D), lambda i: (0, 0)),
        ],
        out_specs=pl.BlockSpec((CB, 3, D), lambda i: (i, 0, 0)),
        out_shape=jax.ShapeDtypeStruct((V, 3, D), jnp.float32),
    )(chain_table, combo)
    return m3.reshape(V * 3, D)


def _sc_combine(seq_table, m_tab, fidx_flat, N, T, D, tok_w, K):
    L = 16
    nvec = D // L
    nchunk = tok_w // K
    mesh = plsc.VectorSubcoreMesh(core_axis_name="c", subcore_axis_name="s")
    num_cores = 2

    @functools.partial(
        pl.kernel,
        mesh=mesh,
        out_type=jax.ShapeDtypeStruct((N, D), jnp.float32),
        scratch_types=[
            pltpu.VMEM((tok_w,), jnp.int32),
            pltpu.VMEM((2, K, D), jnp.float32),
            pltpu.VMEM((2, K, D), jnp.float32),
            pltpu.SemaphoreType.DMA,
            pltpu.SemaphoreType.DMA,
            pltpu.SemaphoreType.DMA,
        ],
    )
    def sc_kernel(seq_hbm, m_hbm, fidx_hbm, out_hbm, idx_v, seq_v, acc_v,
                  sem_g, sem_s, sem_w):
        wid = lax.axis_index("s") * num_cores + lax.axis_index("c")
        base = wid * tok_w
        t0 = lax.rem(base, T)
        pltpu.sync_copy(fidx_hbm.at[pl.ds(base, tok_w)], idx_v)

        gets = [None] * nchunk
        seqs = [None] * nchunk
        puts = [None] * nchunk

        def start(c):
            bk = c % 2
            gets[c] = pltpu.async_copy(
                m_hbm.at[idx_v.at[pl.ds(c * K, K)]], acc_v.at[bk], sem_g)
            seqs[c] = pltpu.async_copy(
                seq_hbm.at[pl.ds(t0 + c * K, K)], seq_v.at[bk], sem_s)

        start(0)
        for c in range(nchunk):
            bk = c % 2
            if c + 1 < nchunk:
                if c >= 1:
                    puts[c - 1].wait()
                start(c + 1)
            gets[c].wait()
            seqs[c].wait()

            def row_body(i, _, bk=bk):
                def vec_body(j, _):
                    sl = pl.ds(j * L, L)
                    acc_v[bk, i, sl] = acc_v[bk, i, sl] + seq_v[bk, i, sl]
                    return 0

                lax.fori_loop(0, nvec, vec_body, 0, unroll=8)
                return 0

            lax.fori_loop(0, K, row_body, 0)
            puts[c] = pltpu.async_copy(
                acc_v.at[bk], out_hbm.at[pl.ds(base + c * K, K)], sem_w)
        puts[nchunk - 2].wait()
        puts[nchunk - 1].wait()

    return sc_kernel(seq_table, m_tab, fidx_flat)


def kernel(input_ids, rel_ids, seq_table, chain_table, depth_table, role_table):
    Bn, Tn = input_ids.shape
    Tseq, D = seq_table.shape
    V = chain_table.shape[0]
    N = Bn * Tn

    role3 = jnp.concatenate(
        [role_table[0:1], role_table[2:3], role_table[3:4]], axis=0)
    combo = 0.3 * depth_table[0:1] + 0.2 * role3

    fidx = _fused_indices(input_ids, rel_ids, V)
    m_tab = _m_table(chain_table, combo)

    num_workers = 32
    tok_w = N // num_workers
    K = 16
    out = _sc_combine(
        seq_table, m_tab, fidx.reshape(N), N, Tn, D, tok_w, K)
    return out.reshape(Bn, Tn, D)
